# baseline (speedup 1.0000x reference)
import functools

import jax
import jax.numpy as jnp
from jax import lax
from jax.experimental import pallas as pl
from jax.experimental.pallas import tpu as pltpu
from jax.experimental.pallas import tpu_sc as plsc

N_NODES = 10000
N_EDGES = 160000
IN_FEAT = 128
OUT_FEAT = 128
D_EDGE = 384
GRID_SIZE = 4
SPLINE_ORDER = 3
COEFFS = GRID_SIZE + SPLINE_ORDER

NC = 2
NS = 16
NW = NC * NS

CH = 80
E_PAD = 163840
CHUNKS = E_PAD // CH
CPT = CHUNKS // NW
ACC_ROWS = 10240
ROWS_PER_SUB = ACC_ROWS // NS
SLAB_CHUNKS = [(z * CH, CH) for z in range(ROWS_PER_SUB // CH)]
TRASH = N_NODES

_H = 2.0 / GRID_SIZE
_KNOTS = [(-SPLINE_ORDER + j) * _H - 1.0 for j in range(GRID_SIZE + 2 * SPLINE_ORDER + 1)]


def _mm_bias_body(x_ref, w_ref, b_ref, o_ref):
    o_ref[...] = (
        jnp.dot(x_ref[...], w_ref[...], preferred_element_type=jnp.float32)
        + b_ref[...]
    )


def _edge_transform(edge_attr, w_t, b2d):
    BE = 1280
    n_real = N_EDGES // BE
    grid = E_PAD // BE
    return pl.pallas_call(
        _mm_bias_body,
        grid=(grid,),
        in_specs=[
            pl.BlockSpec((BE, D_EDGE), lambda i: (jnp.minimum(i, n_real - 1), 0)),
            pl.BlockSpec((D_EDGE, OUT_FEAT), lambda i: (0, 0)),
            pl.BlockSpec((1, OUT_FEAT), lambda i: (0, 0)),
        ],
        out_specs=pl.BlockSpec((BE, OUT_FEAT), lambda i: (i, 0)),
        out_shape=jax.ShapeDtypeStruct((E_PAD, OUT_FEAT), jnp.float32),
    )(edge_attr, w_t, b2d)


def _node_transform(x, w_t, b2d):
    BN = 400
    grid = N_NODES // BN
    return pl.pallas_call(
        _mm_bias_body,
        grid=(grid,),
        in_specs=[
            pl.BlockSpec((BN, IN_FEAT), lambda i: (i, 0)),
            pl.BlockSpec((IN_FEAT, OUT_FEAT), lambda i: (0, 0)),
            pl.BlockSpec((1, OUT_FEAT), lambda i: (0, 0)),
        ],
        out_specs=pl.BlockSpec((BN, OUT_FEAT), lambda i: (i, 0)),
        out_shape=jax.ShapeDtypeStruct((N_NODES, OUT_FEAT), jnp.float32),
    )(x, w_t, b2d)


def _sc_message_body(xt_hbm, ea_hbm, row_hbm, col_hbm, out_hbm,
                     r0, r1, r2, r3, c0, c1, c2, c3,
                     gx0, gx1, ea0, ea1, acc,
                     is0, is1, is2, is3, gs0, gs1, ls0, ls1, ss0, ss1):
    cid = lax.axis_index("c")
    sid = lax.axis_index("s")
    wid = cid * NS + sid
    base = wid * CPT

    R = (r0, r1, r2, r3)
    C = (c0, c1, c2, c3)
    IS = (is0, is1, is2, is3)
    GX = (gx0, gx1)
    EA = (ea0, ea1)
    GS = (gs0, gs1)
    LS = (ls0, ls1)
    SS = (ss0, ss1)

    def _zero_row(i, carry):
        for j in range(8):
            ea0[i, pl.ds(j * 16, 16)] = jnp.zeros((16,), jnp.float32)
        return carry

    lax.fori_loop(0, CH, _zero_row, 0)
    slab = sid * ROWS_PER_SUB
    for off, sz in SLAB_CHUNKS:
        pltpu.sync_copy(ea0.at[pl.ds(0, sz)], acc.at[pl.ds(slab + off, sz)])
    plsc.subcore_barrier()

    def issue_idx(c, s):
        pltpu.async_copy(row_hbm.at[pl.ds(c * CH, CH)], R[s], IS[s])
        pltpu.async_copy(col_hbm.at[pl.ds(c * CH, CH)], C[s], IS[s])

    def wait_idx(s):
        pltpu.make_async_copy(row_hbm.at[pl.ds(0, CH)], R[s], IS[s]).wait()
        pltpu.make_async_copy(col_hbm.at[pl.ds(0, CH)], C[s], IS[s]).wait()

    def issue_ea(c, b):
        pltpu.async_copy(ea_hbm.at[pl.ds(c * CH, CH)], EA[b], LS[b])

    def wait_ea(b):
        pltpu.make_async_copy(ea_hbm.at[pl.ds(0, CH)], EA[b], LS[b]).wait()

    def issue_gather(s, b):
        pltpu.async_copy(xt_hbm.at[R[s]], GX[b], GS[b])

    def wait_gather(b):
        pltpu.make_async_copy(xt_hbm.at[R[0]], GX[b], GS[b]).wait()

    def wait_scatter(b):
        pltpu.make_async_copy(EA[b], acc.at[C[0]], SS[b]).wait()

    def mul_scatter(b, s):
        ea_b, gx_b = EA[b], GX[b]

        def _mul2(i, carry2):
            r = 2 * i
            for rr in (r, r + 1):
                for j in range(8):
                    sl = pl.ds(j * 16, 16)
                    ea_b[rr, sl] = ea_b[rr, sl] * gx_b[rr, sl]
            return carry2

        lax.fori_loop(0, CH // 2, _mul2, 0)
        pltpu.async_copy(ea_b, acc.at[C[s]], SS[b], add=True)

    def step(c, b, s, first=False, pf1=True, pf2=True):
        if pf1:
            wait_idx((s + 1) % 4)
        wait_gather(b)
        wait_ea(b)
        if pf1:
            issue_gather((s + 1) % 4, 1 - b)
        if not first:
            wait_scatter(1 - b)
        if pf1:
            issue_ea(c + 1, 1 - b)
        if pf2:
            issue_idx(c + 2, (s + 2) % 4)
        mul_scatter(b, s)

    issue_idx(base, 0)
    issue_idx(base + 1, 1)
    wait_idx(0)
    issue_gather(0, 0)
    issue_ea(base, 0)
    step(base, 0, 0, first=True)

    def _quad(t, carry):
        c = base + 1 + 4 * t
        step(c, 1, 1)
        step(c + 1, 0, 2)
        step(c + 2, 1, 3)
        step(c + 3, 0, 0)
        return carry

    lax.fori_loop(0, (CPT - 4) // 4, _quad, 0)

    step(base + CPT - 3, 1, 1)
    step(base + CPT - 2, 0, 2, pf2=False)
    step(base + CPT - 1, 1, 3, pf1=False, pf2=False)
    wait_scatter(1)

    plsc.subcore_barrier()

    for off, sz in SLAB_CHUNKS:
        pltpu.sync_copy(acc.at[pl.ds(slab + off, sz)], ea0.at[pl.ds(0, sz)])
        pltpu.sync_copy(ea0.at[pl.ds(0, sz)],
                        out_hbm.at[pl.ds(cid * ACC_ROWS + slab + off, sz)])


@functools.cache
def _sc_message():
    return pl.kernel(
        _sc_message_body,
        out_type=jax.ShapeDtypeStruct((NC * ACC_ROWS, OUT_FEAT), jnp.float32),
        mesh=plsc.VectorSubcoreMesh(core_axis_name="c", subcore_axis_name="s",
                                    num_cores=NC, num_subcores=NS),
        scratch_types=(
            [pltpu.VMEM((CH,), jnp.int32) for _ in range(8)]
            + [pltpu.VMEM((CH, OUT_FEAT), jnp.float32) for _ in range(4)]
            + [pltpu.VMEM_SHARED((ACC_ROWS, OUT_FEAT), jnp.float32)]
            + [pltpu.SemaphoreType.DMA for _ in range(10)]
        ),
    )


def _kan_body(p0_ref, p1_ref, bw_ref, sw_ref, o_ref):
    a = p0_ref[...] + p1_ref[...]
    sig = 1.0 / (1.0 + jnp.exp(-a))
    out = jnp.dot(a * sig, bw_ref[...], preferred_element_type=jnp.float32)

    g = _KNOTS
    bases = [
        jnp.logical_and(a >= g[j], a < g[j + 1]).astype(jnp.float32)
        for j in range(len(g) - 1)
    ]
    for k in range(1, SPLINE_ORDER + 1):
        nxt = []
        for j in range(len(bases) - 1):
            left = (a - g[j]) * (1.0 / (g[j + k] - g[j]))
            right = (g[j + k + 1] - a) * (1.0 / (g[j + k + 1] - g[j + 1]))
            nxt.append(left * bases[j] + right * bases[j + 1])
        bases = nxt

    for c in range(COEFFS):
        out += jnp.dot(bases[c], sw_ref[c], preferred_element_type=jnp.float32)
    o_ref[...] = out


def _kan(partial, bw_t, scaled_t):
    BN = 80
    grid = N_NODES // BN
    half = ACC_ROWS // BN
    return pl.pallas_call(
        _kan_body,
        grid=(grid,),
        in_specs=[
            pl.BlockSpec((BN, OUT_FEAT), lambda i: (i, 0)),
            pl.BlockSpec((BN, OUT_FEAT), lambda i: (i + half, 0)),
            pl.BlockSpec((OUT_FEAT, OUT_FEAT), lambda i: (0, 0)),
            pl.BlockSpec((COEFFS, OUT_FEAT, OUT_FEAT), lambda i: (0, 0, 0)),
        ],
        out_specs=pl.BlockSpec((BN, OUT_FEAT), lambda i: (i, 0)),
        out_shape=jax.ShapeDtypeStruct((N_NODES, OUT_FEAT), jnp.float32),
    )(partial, partial, bw_t, scaled_t)


def kernel(x, edge_index, edge_attr, W_edge, b_edge, W_node, b_node,
           base_weight, spline_weight, spline_scaler):
    w_edge_t = W_edge.T
    w_node_t = W_node.T
    b_edge2d = b_edge[None, :]
    b_node2d = b_node[None, :]
    bw_t = base_weight.T
    scaled_t = (spline_weight * spline_scaler[:, :, None]).transpose(2, 1, 0)

    pad = E_PAD - N_EDGES
    row_p = jnp.concatenate([edge_index[0], jnp.zeros((pad,), jnp.int32)])
    col_p = jnp.concatenate([edge_index[1], jnp.full((pad,), TRASH, jnp.int32)])

    x_t = _node_transform(x, w_node_t, b_node2d)
    ea_t = _edge_transform(edge_attr, w_edge_t, b_edge2d)
    partial = _sc_message()(x_t, ea_t, row_p, col_p)
    return _kan(partial, bw_t, scaled_t)


# speedup vs baseline: 1.0064x; 1.0064x over previous
import functools

import jax
import jax.numpy as jnp
from jax import lax
from jax.experimental import pallas as pl
from jax.experimental.pallas import tpu as pltpu
from jax.experimental.pallas import tpu_sc as plsc

N_NODES = 10000
N_EDGES = 160000
IN_FEAT = 128
OUT_FEAT = 128
D_EDGE = 384
GRID_SIZE = 4
SPLINE_ORDER = 3
COEFFS = GRID_SIZE + SPLINE_ORDER

NC = 2
NS = 16
NW = NC * NS

CH = 80
E_PAD = 163840
CHUNKS = E_PAD // CH
N_SLICES = 2
E_SLICE = E_PAD // N_SLICES
CPT = CHUNKS // N_SLICES // NW
ACC_ROWS = 10240
ROWS_PER_SUB = ACC_ROWS // NS
SLAB_CHUNKS = [(z * CH, CH) for z in range(ROWS_PER_SUB // CH)]
TRASH = N_NODES

_H = 2.0 / GRID_SIZE
_KNOTS = [(-SPLINE_ORDER + j) * _H - 1.0 for j in range(GRID_SIZE + 2 * SPLINE_ORDER + 1)]


def _mm_bias_body(x_ref, w_ref, b_ref, o_ref):
    o_ref[...] = (
        jnp.dot(x_ref[...], w_ref[...], preferred_element_type=jnp.float32)
        + b_ref[...]
    )


def _edge_transform(edge_attr, w_t, b2d, start_blk, nblk, max_blk):
    BE = 1280
    return pl.pallas_call(
        _mm_bias_body,
        grid=(nblk,),
        in_specs=[
            pl.BlockSpec(
                (BE, D_EDGE),
                lambda i: (jnp.minimum(start_blk + i, max_blk), 0)),
            pl.BlockSpec((D_EDGE, OUT_FEAT), lambda i: (0, 0)),
            pl.BlockSpec((1, OUT_FEAT), lambda i: (0, 0)),
        ],
        out_specs=pl.BlockSpec((BE, OUT_FEAT), lambda i: (i, 0)),
        out_shape=jax.ShapeDtypeStruct((nblk * BE, OUT_FEAT), jnp.float32),
    )(edge_attr, w_t, b2d)


def _node_transform(x, w_t, b2d):
    BN = 400
    grid = N_NODES // BN
    return pl.pallas_call(
        _mm_bias_body,
        grid=(grid,),
        in_specs=[
            pl.BlockSpec((BN, IN_FEAT), lambda i: (i, 0)),
            pl.BlockSpec((IN_FEAT, OUT_FEAT), lambda i: (0, 0)),
            pl.BlockSpec((1, OUT_FEAT), lambda i: (0, 0)),
        ],
        out_specs=pl.BlockSpec((BN, OUT_FEAT), lambda i: (i, 0)),
        out_shape=jax.ShapeDtypeStruct((N_NODES, OUT_FEAT), jnp.float32),
    )(x, w_t, b2d)


def _sc_message_body_factory(idx_off):
    def _sc_message_body(xt_hbm, ea_hbm, row_hbm, col_hbm, out_hbm,
                         r0, r1, r2, r3, c0, c1, c2, c3,
                         gx0, gx1, ea0, ea1, acc,
                         is0, is1, is2, is3, gs0, gs1, ls0, ls1, ss0, ss1):
        return _sc_message_impl(
            idx_off, xt_hbm, ea_hbm, row_hbm, col_hbm, out_hbm,
            r0, r1, r2, r3, c0, c1, c2, c3, gx0, gx1, ea0, ea1, acc,
            is0, is1, is2, is3, gs0, gs1, ls0, ls1, ss0, ss1)
    return _sc_message_body


def _sc_message_impl(idx_off, xt_hbm, ea_hbm, row_hbm, col_hbm, out_hbm,
                     r0, r1, r2, r3, c0, c1, c2, c3,
                     gx0, gx1, ea0, ea1, acc,
                     is0, is1, is2, is3, gs0, gs1, ls0, ls1, ss0, ss1):
    cid = lax.axis_index("c")
    sid = lax.axis_index("s")
    wid = cid * NS + sid
    base = wid * CPT

    R = (r0, r1, r2, r3)
    C = (c0, c1, c2, c3)
    IS = (is0, is1, is2, is3)
    GX = (gx0, gx1)
    EA = (ea0, ea1)
    GS = (gs0, gs1)
    LS = (ls0, ls1)
    SS = (ss0, ss1)

    def _zero_row(i, carry):
        for j in range(8):
            ea0[i, pl.ds(j * 16, 16)] = jnp.zeros((16,), jnp.float32)
        return carry

    lax.fori_loop(0, CH, _zero_row, 0)
    slab = sid * ROWS_PER_SUB
    for off, sz in SLAB_CHUNKS:
        pltpu.sync_copy(ea0.at[pl.ds(0, sz)], acc.at[pl.ds(slab + off, sz)])
    plsc.subcore_barrier()

    def issue_idx(c, s):
        g = (idx_off + c) * CH
        pltpu.async_copy(row_hbm.at[pl.ds(g, CH)], R[s], IS[s])
        pltpu.async_copy(col_hbm.at[pl.ds(g, CH)], C[s], IS[s])

    def wait_idx(s):
        pltpu.make_async_copy(row_hbm.at[pl.ds(0, CH)], R[s], IS[s]).wait()
        pltpu.make_async_copy(col_hbm.at[pl.ds(0, CH)], C[s], IS[s]).wait()

    def issue_ea(c, b):
        pltpu.async_copy(ea_hbm.at[pl.ds(c * CH, CH)], EA[b], LS[b])

    def wait_ea(b):
        pltpu.make_async_copy(ea_hbm.at[pl.ds(0, CH)], EA[b], LS[b]).wait()

    def issue_gather(s, b):
        pltpu.async_copy(xt_hbm.at[R[s]], GX[b], GS[b])

    def wait_gather(b):
        pltpu.make_async_copy(xt_hbm.at[R[0]], GX[b], GS[b]).wait()

    def wait_scatter(b):
        pltpu.make_async_copy(EA[b], acc.at[C[0]], SS[b]).wait()

    def mul_scatter(b, s):
        ea_b, gx_b = EA[b], GX[b]

        def _mul2(i, carry2):
            r = 2 * i
            for rr in (r, r + 1):
                for j in range(8):
                    sl = pl.ds(j * 16, 16)
                    ea_b[rr, sl] = ea_b[rr, sl] * gx_b[rr, sl]
            return carry2

        lax.fori_loop(0, CH // 2, _mul2, 0)
        pltpu.async_copy(ea_b, acc.at[C[s]], SS[b], add=True)

    def step(c, b, s, first=False, pf1=True, pf2=True):
        if pf1:
            wait_idx((s + 1) % 4)
        wait_gather(b)
        wait_ea(b)
        if pf1:
            issue_gather((s + 1) % 4, 1 - b)
        if not first:
            wait_scatter(1 - b)
        if pf1:
            issue_ea(c + 1, 1 - b)
        if pf2:
            issue_idx(c + 2, (s + 2) % 4)
        mul_scatter(b, s)

    issue_idx(base, 0)
    issue_idx(base + 1, 1)
    wait_idx(0)
    issue_gather(0, 0)
    issue_ea(base, 0)
    step(base, 0, 0, first=True)

    def _quad(t, carry):
        c = base + 1 + 4 * t
        step(c, 1, 1)
        step(c + 1, 0, 2)
        step(c + 2, 1, 3)
        step(c + 3, 0, 0)
        return carry

    lax.fori_loop(0, (CPT - 4) // 4, _quad, 0)

    step(base + CPT - 3, 1, 1)
    step(base + CPT - 2, 0, 2, pf2=False)
    step(base + CPT - 1, 1, 3, pf1=False, pf2=False)
    wait_scatter(1)

    plsc.subcore_barrier()

    for off, sz in SLAB_CHUNKS:
        pltpu.sync_copy(acc.at[pl.ds(slab + off, sz)], ea0.at[pl.ds(0, sz)])
        pltpu.sync_copy(ea0.at[pl.ds(0, sz)],
                        out_hbm.at[pl.ds(cid * ACC_ROWS + slab + off, sz)])


@functools.cache
def _sc_message(slice_idx):
    return pl.kernel(
        _sc_message_body_factory(slice_idx * (E_SLICE // CH)),
        out_type=jax.ShapeDtypeStruct((NC * ACC_ROWS, OUT_FEAT), jnp.float32),
        mesh=plsc.VectorSubcoreMesh(core_axis_name="c", subcore_axis_name="s",
                                    num_cores=NC, num_subcores=NS),
        scratch_types=(
            [pltpu.VMEM((CH,), jnp.int32) for _ in range(8)]
            + [pltpu.VMEM((CH, OUT_FEAT), jnp.float32) for _ in range(4)]
            + [pltpu.VMEM_SHARED((ACC_ROWS, OUT_FEAT), jnp.float32)]
            + [pltpu.SemaphoreType.DMA for _ in range(10)]
        ),
    )


def _kan_body(p0_ref, p1_ref, p2_ref, p3_ref, bw_ref, sw_ref, o_ref):
    a = (p0_ref[...] + p1_ref[...]) + (p2_ref[...] + p3_ref[...])
    sig = 1.0 / (1.0 + jnp.exp(-a))
    out = jnp.dot(a * sig, bw_ref[...], preferred_element_type=jnp.float32)

    g = _KNOTS
    bases = [
        jnp.logical_and(a >= g[j], a < g[j + 1]).astype(jnp.float32)
        for j in range(len(g) - 1)
    ]
    for k in range(1, SPLINE_ORDER + 1):
        nxt = []
        for j in range(len(bases) - 1):
            left = (a - g[j]) * (1.0 / (g[j + k] - g[j]))
            right = (g[j + k + 1] - a) * (1.0 / (g[j + k + 1] - g[j + 1]))
            nxt.append(left * bases[j] + right * bases[j + 1])
        bases = nxt

    for c in range(COEFFS):
        out += jnp.dot(bases[c], sw_ref[c], preferred_element_type=jnp.float32)
    o_ref[...] = out


def _kan(part_a, part_b, bw_t, scaled_t):
    BN = 80
    grid = N_NODES // BN
    half = ACC_ROWS // BN
    return pl.pallas_call(
        _kan_body,
        grid=(grid,),
        in_specs=[
            pl.BlockSpec((BN, OUT_FEAT), lambda i: (i, 0)),
            pl.BlockSpec((BN, OUT_FEAT), lambda i: (i + half, 0)),
            pl.BlockSpec((BN, OUT_FEAT), lambda i: (i, 0)),
            pl.BlockSpec((BN, OUT_FEAT), lambda i: (i + half, 0)),
            pl.BlockSpec((OUT_FEAT, OUT_FEAT), lambda i: (0, 0)),
            pl.BlockSpec((COEFFS, OUT_FEAT, OUT_FEAT), lambda i: (0, 0, 0)),
        ],
        out_specs=pl.BlockSpec((BN, OUT_FEAT), lambda i: (i, 0)),
        out_shape=jax.ShapeDtypeStruct((N_NODES, OUT_FEAT), jnp.float32),
    )(part_a, part_a, part_b, part_b, bw_t, scaled_t)


def kernel(x, edge_index, edge_attr, W_edge, b_edge, W_node, b_node,
           base_weight, spline_weight, spline_scaler):
    w_edge_t = W_edge.T
    w_node_t = W_node.T
    b_edge2d = b_edge[None, :]
    b_node2d = b_node[None, :]
    bw_t = base_weight.T
    scaled_t = (spline_weight * spline_scaler[:, :, None]).transpose(2, 1, 0)

    pad = E_PAD - N_EDGES
    row_p = jnp.concatenate([edge_index[0], jnp.zeros((pad,), jnp.int32)])
    col_p = jnp.concatenate([edge_index[1], jnp.full((pad,), TRASH, jnp.int32)])

    x_t = _node_transform(x, w_node_t, b_node2d)

    BE = 1280
    nblk = E_SLICE // BE
    max_blk = N_EDGES // BE - 1
    ea_a = _edge_transform(edge_attr, w_edge_t, b_edge2d, 0, nblk, max_blk)
    part_a = _sc_message(0)(x_t, ea_a, row_p, col_p)
    ea_b = _edge_transform(edge_attr, w_edge_t, b_edge2d, nblk, nblk, max_blk)
    part_b = _sc_message(1)(x_t, ea_b, row_p, col_p)
    return _kan(part_a, part_b, bw_t, scaled_t)


# speedup vs baseline: 1.0913x; 1.0844x over previous
import functools

import jax
import jax.numpy as jnp
from jax import lax
from jax.experimental import pallas as pl
from jax.experimental.pallas import tpu as pltpu
from jax.experimental.pallas import tpu_sc as plsc

N_NODES = 10000
N_EDGES = 160000
IN_FEAT = 128
OUT_FEAT = 128
D_EDGE = 384
GRID_SIZE = 4
SPLINE_ORDER = 3
COEFFS = GRID_SIZE + SPLINE_ORDER

NC = 2
NS = 16
NW = NC * NS

CH = 80
E_PAD = 163840
CHUNKS = E_PAD // CH
N_SLICES = 2
E_SLICE = E_PAD // N_SLICES
CPT = CHUNKS // N_SLICES // NW
ACC_ROWS = 10240
ROWS_PER_SUB = ACC_ROWS // NS
SLAB_CHUNKS = [(z * CH, CH) for z in range(ROWS_PER_SUB // CH)]
TRASH = N_NODES

_H = 2.0 / GRID_SIZE
_KNOTS = [(-SPLINE_ORDER + j) * _H - 1.0 for j in range(GRID_SIZE + 2 * SPLINE_ORDER + 1)]


def _mm_bias_body(x_ref, w_ref, b_ref, o_ref):
    o_ref[...] = (
        jnp.dot(x_ref[...], w_ref[...], preferred_element_type=jnp.float32)
        + b_ref[...]
    )


def _edge_transform(edge_attr, w_t, b2d, start_blk, nblk, max_blk):
    BE = 1280
    return pl.pallas_call(
        _mm_bias_body,
        grid=(nblk,),
        in_specs=[
            pl.BlockSpec(
                (BE, D_EDGE),
                lambda i: (jnp.minimum(start_blk + i, max_blk), 0)),
            pl.BlockSpec((D_EDGE, OUT_FEAT), lambda i: (0, 0)),
            pl.BlockSpec((1, OUT_FEAT), lambda i: (0, 0)),
        ],
        out_specs=pl.BlockSpec((BE, OUT_FEAT), lambda i: (i, 0)),
        out_shape=jax.ShapeDtypeStruct((nblk * BE, OUT_FEAT), jnp.float32),
    )(edge_attr, w_t, b2d)


def _node_transform(x, w_t, b2d):
    BN = 400
    grid = N_NODES // BN
    return pl.pallas_call(
        _mm_bias_body,
        grid=(grid,),
        in_specs=[
            pl.BlockSpec((BN, IN_FEAT), lambda i: (i, 0)),
            pl.BlockSpec((IN_FEAT, OUT_FEAT), lambda i: (0, 0)),
            pl.BlockSpec((1, OUT_FEAT), lambda i: (0, 0)),
        ],
        out_specs=pl.BlockSpec((BN, OUT_FEAT), lambda i: (i, 0)),
        out_shape=jax.ShapeDtypeStruct((N_NODES, OUT_FEAT), jnp.float32),
    )(x, w_t, b2d)


def _sc_message_body_factory(idx_off):
    def _sc_message_body(xt_hbm, ea_hbm, row_hbm, col_hbm, out0_hbm, out1_hbm,
                         r0, r1, r2, r3, c0, c1, c2, c3,
                         gx0, gx1, ea0, ea1, acc,
                         is0, is1, is2, is3, gs0, gs1, ls0, ls1, ss0, ss1):
        return _sc_message_impl(
            idx_off, xt_hbm, ea_hbm, row_hbm, col_hbm, out0_hbm, out1_hbm,
            r0, r1, r2, r3, c0, c1, c2, c3, gx0, gx1, ea0, ea1, acc,
            is0, is1, is2, is3, gs0, gs1, ls0, ls1, ss0, ss1)
    return _sc_message_body


def _sc_message_impl(idx_off, xt_hbm, ea_hbm, row_hbm, col_hbm,
                     out0_hbm, out1_hbm,
                     r0, r1, r2, r3, c0, c1, c2, c3,
                     gx0, gx1, ea0, ea1, acc,
                     is0, is1, is2, is3, gs0, gs1, ls0, ls1, ss0, ss1):
    cid = lax.axis_index("c")
    sid = lax.axis_index("s")
    wid = cid * NS + sid
    base = wid * CPT

    R = (r0, r1, r2, r3)
    C = (c0, c1, c2, c3)
    IS = (is0, is1, is2, is3)
    GX = (gx0, gx1)
    EA = (ea0, ea1)
    GS = (gs0, gs1)
    LS = (ls0, ls1)
    SS = (ss0, ss1)

    def _zero_row(i, carry):
        for j in range(8):
            ea0[i, pl.ds(j * 16, 16)] = jnp.zeros((16,), jnp.float32)
        return carry

    lax.fori_loop(0, CH, _zero_row, 0)
    slab = sid * ROWS_PER_SUB
    for off, sz in SLAB_CHUNKS:
        pltpu.sync_copy(ea0.at[pl.ds(0, sz)], acc.at[pl.ds(slab + off, sz)])
    plsc.subcore_barrier()

    def issue_idx(c, s):
        g = (idx_off + c) * CH
        pltpu.async_copy(row_hbm.at[pl.ds(g, CH)], R[s], IS[s])
        pltpu.async_copy(col_hbm.at[pl.ds(g, CH)], C[s], IS[s])

    def wait_idx(s):
        pltpu.make_async_copy(row_hbm.at[pl.ds(0, CH)], R[s], IS[s]).wait()
        pltpu.make_async_copy(col_hbm.at[pl.ds(0, CH)], C[s], IS[s]).wait()

    def issue_ea(c, b):
        pltpu.async_copy(ea_hbm.at[pl.ds(c * CH, CH)], EA[b], LS[b])

    def wait_ea(b):
        pltpu.make_async_copy(ea_hbm.at[pl.ds(0, CH)], EA[b], LS[b]).wait()

    def issue_gather(s, b):
        pltpu.async_copy(xt_hbm.at[R[s]], GX[b], GS[b])

    def wait_gather(b):
        pltpu.make_async_copy(xt_hbm.at[R[0]], GX[b], GS[b]).wait()

    def wait_scatter(b):
        pltpu.make_async_copy(EA[b], acc.at[C[0]], SS[b]).wait()

    def mul_scatter(b, s):
        ea_b, gx_b = EA[b], GX[b]

        def _mul2(i, carry2):
            r = 2 * i
            for rr in (r, r + 1):
                for j in range(8):
                    sl = pl.ds(j * 16, 16)
                    ea_b[rr, sl] = ea_b[rr, sl] * gx_b[rr, sl]
            return carry2

        lax.fori_loop(0, CH // 2, _mul2, 0)
        pltpu.async_copy(ea_b, acc.at[C[s]], SS[b], add=True)

    def step(c, b, s, first=False, pf1=True, pf2=True):
        if pf1:
            wait_idx((s + 1) % 4)
        wait_gather(b)
        wait_ea(b)
        if pf1:
            issue_gather((s + 1) % 4, 1 - b)
        if not first:
            wait_scatter(1 - b)
        if pf1:
            issue_ea(c + 1, 1 - b)
        if pf2:
            issue_idx(c + 2, (s + 2) % 4)
        mul_scatter(b, s)

    issue_idx(base, 0)
    issue_idx(base + 1, 1)
    wait_idx(0)
    issue_gather(0, 0)
    issue_ea(base, 0)
    step(base, 0, 0, first=True)

    def _quad(t, carry):
        c = base + 1 + 4 * t
        step(c, 1, 1)
        step(c + 1, 0, 2)
        step(c + 2, 1, 3)
        step(c + 3, 0, 0)
        return carry

    lax.fori_loop(0, (CPT - 4) // 4, _quad, 0)

    step(base + CPT - 3, 1, 1)
    step(base + CPT - 2, 0, 2, pf2=False)
    step(base + CPT - 1, 1, 3, pf1=False, pf2=False)
    wait_scatter(1)

    plsc.subcore_barrier()

    for off, sz in SLAB_CHUNKS:
        pltpu.sync_copy(acc.at[pl.ds(slab + off, sz)], ea0.at[pl.ds(0, sz)])
        @pl.when(cid == 0)
        def _copy0():
            pltpu.sync_copy(ea0.at[pl.ds(0, sz)],
                            out0_hbm.at[pl.ds(slab + off, sz)])

        @pl.when(cid == 1)
        def _copy1():
            pltpu.sync_copy(ea0.at[pl.ds(0, sz)],
                            out1_hbm.at[pl.ds(slab + off, sz)])


@functools.cache
def _sc_message(slice_idx):
    return pl.kernel(
        _sc_message_body_factory(slice_idx * (E_SLICE // CH)),
        out_type=(jax.ShapeDtypeStruct((ACC_ROWS, OUT_FEAT), jnp.float32),
                  jax.ShapeDtypeStruct((ACC_ROWS, OUT_FEAT), jnp.float32)),
        mesh=plsc.VectorSubcoreMesh(core_axis_name="c", subcore_axis_name="s",
                                    num_cores=NC, num_subcores=NS),
        scratch_types=(
            [pltpu.VMEM((CH,), jnp.int32) for _ in range(8)]
            + [pltpu.VMEM((CH, OUT_FEAT), jnp.float32) for _ in range(4)]
            + [pltpu.VMEM_SHARED((ACC_ROWS, OUT_FEAT), jnp.float32)]
            + [pltpu.SemaphoreType.DMA for _ in range(10)]
        ),
    )


def _kan_body(p0_ref, p1_ref, p2_ref, p3_ref, bw_ref, sw_ref, o_ref):
    a = (p0_ref[...] + p1_ref[...]) + (p2_ref[...] + p3_ref[...])
    sig = 1.0 / (1.0 + jnp.exp(-a))
    out = jnp.dot(a * sig, bw_ref[...], preferred_element_type=jnp.float32)

    g = _KNOTS
    bases = [
        jnp.logical_and(a >= g[j], a < g[j + 1]).astype(jnp.float32)
        for j in range(len(g) - 1)
    ]
    for k in range(1, SPLINE_ORDER + 1):
        nxt = []
        for j in range(len(bases) - 1):
            left = (a - g[j]) * (1.0 / (g[j + k] - g[j]))
            right = (g[j + k + 1] - a) * (1.0 / (g[j + k + 1] - g[j + 1]))
            nxt.append(left * bases[j] + right * bases[j + 1])
        bases = nxt

    for c in range(COEFFS):
        out += jnp.dot(bases[c], sw_ref[c], preferred_element_type=jnp.float32)
    o_ref[...] = out


def _kan(parts, bw_t, scaled_t):
    BN = 400
    grid = N_NODES // BN
    specs = [pl.BlockSpec((BN, OUT_FEAT), lambda i: (i, 0)) for _ in parts]
    return pl.pallas_call(
        _kan_body,
        grid=(grid,),
        in_specs=specs + [
            pl.BlockSpec((OUT_FEAT, OUT_FEAT), lambda i: (0, 0)),
            pl.BlockSpec((COEFFS, OUT_FEAT, OUT_FEAT), lambda i: (0, 0, 0)),
        ],
        out_specs=pl.BlockSpec((BN, OUT_FEAT), lambda i: (i, 0)),
        out_shape=jax.ShapeDtypeStruct((N_NODES, OUT_FEAT), jnp.float32),
    )(*parts, bw_t, scaled_t)


def kernel(x, edge_index, edge_attr, W_edge, b_edge, W_node, b_node,
           base_weight, spline_weight, spline_scaler):
    w_edge_t = W_edge.T
    w_node_t = W_node.T
    b_edge2d = b_edge[None, :]
    b_node2d = b_node[None, :]
    bw_t = base_weight.T
    scaled_t = (spline_weight * spline_scaler[:, :, None]).transpose(2, 1, 0)

    pad = E_PAD - N_EDGES
    row_p = jnp.concatenate([edge_index[0], jnp.zeros((pad,), jnp.int32)])
    col_p = jnp.concatenate([edge_index[1], jnp.full((pad,), TRASH, jnp.int32)])

    x_t = _node_transform(x, w_node_t, b_node2d)

    BE = 1280
    nblk = E_SLICE // BE
    max_blk = N_EDGES // BE - 1
    ea_a = _edge_transform(edge_attr, w_edge_t, b_edge2d, 0, nblk, max_blk)
    pa0, pa1 = _sc_message(0)(x_t, ea_a, row_p, col_p)
    ea_b = _edge_transform(edge_attr, w_edge_t, b_edge2d, nblk, nblk, max_blk)
    pb0, pb1 = _sc_message(1)(x_t, ea_b, row_p, col_p)
    return _kan([pa0, pa1, pb0, pb1], bw_t, scaled_t)
